# Initial kernel scaffold; baseline (speedup 1.0000x reference)
import functools

import jax
import jax.numpy as jnp
from jax import lax
from jax.experimental import pallas as pl
from jax.experimental.pallas import tpu as pltpu
from jax.experimental.pallas import tpu_sc as plsc

NC = 2
NS = 16
L = 16

BN = 400


def _dense_layer1_body(texts, nf, Wt, bt, Wg01, bg01, Wg02, bg02,
                       Wg11, bg11, Wg12, bg12, W0, b0,
                       cur_out, s2_out):
    te = jnp.dot(texts[...], Wt[...], preferred_element_type=jnp.float32) + bt[...]
    t1 = jax.nn.relu(jnp.dot(te, Wg01[...], preferred_element_type=jnp.float32) + bg01[...])
    s1 = jax.nn.sigmoid(jnp.dot(t1, Wg02[...], preferred_element_type=jnp.float32) + bg02[...])
    c = (jnp.dot(nf[...], W0[...], preferred_element_type=jnp.float32) + b0[...]) * s1
    cur_out[...] = c.reshape(cur_out.shape)
    t2 = jax.nn.relu(jnp.dot(te, Wg11[...], preferred_element_type=jnp.float32) + bg11[...])
    s2_out[...] = jax.nn.sigmoid(jnp.dot(t2, Wg12[...], preferred_element_type=jnp.float32) + bg12[...])


def _dense_layer2_body(cur, agg_a, agg_b, deg, s2, W1, b1, cur2_out):
    bn = deg.shape[0]
    rdeg = 1.0 / jnp.maximum(deg[...], 1.0)
    agg = jnp.concatenate([agg_a[...], agg_b[...]], axis=1)
    c = cur[...].reshape(bn, -1)
    h = jax.nn.relu((c + agg * rdeg) * 0.5)
    c2 = (jnp.dot(h, W1[...], preferred_element_type=jnp.float32) + b1[...]) * s2[...]
    cur2_out[...] = c2.reshape(cur2_out.shape)


def _combine_body(cur, agg_a, agg_b, deg, out):
    bn = deg.shape[0]
    rdeg = 1.0 / jnp.maximum(deg[...], 1.0)
    agg = jnp.concatenate([agg_a[...], agg_b[...]], axis=1)
    out[...] = (cur[...].reshape(bn, -1) + agg * rdeg) * 0.5


def _row_spec(bshape):
    nd = len(bshape)
    return pl.BlockSpec(bshape, lambda i, _nd=nd: (i,) + (0,) * (_nd - 1))


def _full_spec(shape):
    nd = len(shape)
    return pl.BlockSpec(shape, lambda i, _nd=nd: (0,) * _nd)


def _make_sc_aggregate(n, e, h):
    hh = h // NC
    ept = e // NS
    ch = 80
    nch = ept // ch
    assert ept % ch == 0 and n % NS == 0 and hh == 128
    rpt = n // NS
    zr = 125
    assert rpt % zr == 0
    degp = ((n + NS * L - 1) // (NS * L)) * NS * L
    rr = degp // NS

    mesh = plsc.VectorSubcoreMesh(core_axis_name="c", subcore_axis_name="s",
                                  num_cores=NC, num_subcores=NS)

    @functools.partial(
        pl.kernel,
        out_type=(
            jax.ShapeDtypeStruct((n, hh), jnp.float32),
            jax.ShapeDtypeStruct((n, hh), jnp.float32),
            jax.ShapeDtypeStruct((degp,), jnp.float32),
        ),
        mesh=mesh,
        scratch_types=[
            pltpu.VMEM((zr, hh), jnp.float32),
            pltpu.VMEM((ch,), jnp.int32),
            pltpu.VMEM((ch,), jnp.int32),
            pltpu.VMEM((ch,), jnp.int32),
            pltpu.VMEM((ch, hh), jnp.float32),
            pltpu.VMEM((degp,), jnp.float32),
            pltpu.VMEM((NS, rr), jnp.float32),
            pltpu.VMEM((rr,), jnp.float32),
            pltpu.VMEM_SHARED((n, hh), jnp.float32),
            pltpu.VMEM_SHARED((NS, degp), jnp.float32),
            pltpu.SemaphoreType.DMA,
        ],
    )
    def sc_aggregate(row_hbm, col_hbm, cur_hbm, agg_a_hbm, agg_b_hbm, deg_hbm,
                     zbuf, idx_row, idx_col, idx2, rows, deg_v, red_buf,
                     deg_out_v, agg_sh, deg_sh, gsem):
        cid = lax.axis_index("c")
        sid = lax.axis_index("s")
        zeros16 = jnp.zeros((L,), jnp.float32)
        ones16 = jnp.ones((L,), jnp.float32)

        def _zrow(i, _):
            for j in range(hh // L):
                zbuf[i, pl.ds(j * L, L)] = zeros16
            return 0
        lax.fori_loop(0, zr, _zrow, 0)

        def _zdeg(i, _):
            deg_v[pl.ds(i * L, L)] = zeros16
            return 0
        lax.fori_loop(0, degp // L, _zdeg, 0)

        for j in range(rpt // zr):
            pltpu.sync_copy(zbuf, agg_sh.at[pl.ds(sid * rpt + j * zr, zr)])
        plsc.subcore_barrier()

        def _chunk(k, _):
            base = sid * ept + k * ch
            pltpu.sync_copy(row_hbm.at[pl.ds(base, ch)], idx_row)
            pltpu.sync_copy(col_hbm.at[pl.ds(base, ch)], idx_col)
            for j in range(ch // L):
                r = idx_row[pl.ds(j * L, L)]
                idx2[pl.ds(j * L, L)] = r + r + cid

            @pl.when(cid == 0)
            def _():
                for j in range(ch // L):
                    c = idx_col[pl.ds(j * L, L)]
                    plsc.addupdate_scatter(deg_v, [c], ones16)

            pltpu.async_copy(cur_hbm.at[idx2], rows, gsem).wait()
            pltpu.sync_copy(rows, agg_sh.at[idx_col], add=True)
            return 0
        lax.fori_loop(0, nch, _chunk, 0)
        plsc.subcore_barrier()

        @pl.when(cid == 0)
        def _():
            pltpu.sync_copy(agg_sh.at[pl.ds(sid * rpt, rpt)],
                            agg_a_hbm.at[pl.ds(sid * rpt, rpt)])
            pltpu.sync_copy(deg_v, deg_sh.at[sid])

        @pl.when(cid == 1)
        def _():
            pltpu.sync_copy(agg_sh.at[pl.ds(sid * rpt, rpt)],
                            agg_b_hbm.at[pl.ds(sid * rpt, rpt)])

        plsc.subcore_barrier()

        @pl.when(cid == 0)
        def _():
            for tt in range(NS):
                pltpu.sync_copy(deg_sh.at[tt, pl.ds(sid * rr, rr)], red_buf.at[tt])

            def _red(v, _):
                acc = red_buf[0, pl.ds(v * L, L)]
                for tt in range(1, NS):
                    acc = acc + red_buf[tt, pl.ds(v * L, L)]
                deg_out_v[pl.ds(v * L, L)] = acc
                return 0
            lax.fori_loop(0, rr // L, _red, 0)
            pltpu.sync_copy(deg_out_v, deg_hbm.at[pl.ds(sid * rr, rr)])

    return sc_aggregate, degp


def kernel(edge_index, node_features, node_texts, Wt, bt, Wg01, bg01, Wg02,
           bg02, Wg11, bg11, Wg12, bg12, W0, b0, W1, b1):
    n, d = node_features.shape
    t = node_texts.shape[1]
    h = W0.shape[1]
    e = edge_index.shape[1]
    assert n % BN == 0
    nblk = n // BN

    row = edge_index[0]
    col = edge_index[1]
    b2 = lambda b: b.reshape(1, -1)

    cur1, s2 = pl.pallas_call(
        _dense_layer1_body,
        grid=(nblk,),
        in_specs=[
            _row_spec((BN, t)), _row_spec((BN, d)),
            _full_spec((t, h)), _full_spec((1, h)),
            _full_spec((h, h)), _full_spec((1, h)),
            _full_spec((h, h)), _full_spec((1, h)),
            _full_spec((h, h)), _full_spec((1, h)),
            _full_spec((h, h)), _full_spec((1, h)),
            _full_spec((d, h)), _full_spec((1, h)),
        ],
        out_specs=[_row_spec((BN, 2, h // 2)), _row_spec((BN, h))],
        out_shape=[
            jax.ShapeDtypeStruct((n, 2, h // 2), jnp.float32),
            jax.ShapeDtypeStruct((n, h), jnp.float32),
        ],
    )(node_texts, node_features, Wt, b2(bt), Wg01, b2(bg01), Wg02, b2(bg02),
      Wg11, b2(bg11), Wg12, b2(bg12), W0, b2(b0))

    sc_aggregate, degp = _make_sc_aggregate(n, e, h)

    agg1_a, agg1_b, deg = sc_aggregate(row, col, cur1.reshape(2 * n, h // 2))
    deg2d = deg[:n].reshape(n, 1)

    cur2 = pl.pallas_call(
        _dense_layer2_body,
        grid=(nblk,),
        in_specs=[
            _row_spec((BN, 2, h // 2)), _row_spec((BN, h // 2)),
            _row_spec((BN, h // 2)), _row_spec((BN, 1)), _row_spec((BN, h)),
            _full_spec((h, h)), _full_spec((1, h)),
        ],
        out_specs=_row_spec((BN, 2, h // 2)),
        out_shape=jax.ShapeDtypeStruct((n, 2, h // 2), jnp.float32),
    )(cur1, agg1_a, agg1_b, deg2d, s2, W1, b2(b1))

    agg2_a, agg2_b, _ = sc_aggregate(row, col, cur2.reshape(2 * n, h // 2))

    out = pl.pallas_call(
        _combine_body,
        grid=(nblk,),
        in_specs=[
            _row_spec((BN, 2, h // 2)), _row_spec((BN, h // 2)),
            _row_spec((BN, h // 2)), _row_spec((BN, 1)),
        ],
        out_specs=_row_spec((BN, h)),
        out_shape=jax.ShapeDtypeStruct((n, h), jnp.float32),
    )(cur2, agg2_a, agg2_b, deg2d)
    return out


# speedup vs baseline: 3.4575x; 3.4575x over previous
import functools

import jax
import jax.numpy as jnp
from jax import lax
from jax.experimental import pallas as pl
from jax.experimental.pallas import tpu as pltpu
from jax.experimental.pallas import tpu_sc as plsc

NC = 2
NS = 16
L = 16

BN = 400


def _dense_layer1_body(texts, nf, Wt, bt, Wg01, bg01, Wg02, bg02,
                       Wg11, bg11, Wg12, bg12, W0, b0,
                       cur_out, s2_out):
    te = jnp.dot(texts[...], Wt[...], preferred_element_type=jnp.float32) + bt[...]
    t1 = jax.nn.relu(jnp.dot(te, Wg01[...], preferred_element_type=jnp.float32) + bg01[...])
    s1 = jax.nn.sigmoid(jnp.dot(t1, Wg02[...], preferred_element_type=jnp.float32) + bg02[...])
    c = (jnp.dot(nf[...], W0[...], preferred_element_type=jnp.float32) + b0[...]) * s1
    cur_out[...] = c.reshape(cur_out.shape)
    t2 = jax.nn.relu(jnp.dot(te, Wg11[...], preferred_element_type=jnp.float32) + bg11[...])
    s2_out[...] = jax.nn.sigmoid(jnp.dot(t2, Wg12[...], preferred_element_type=jnp.float32) + bg12[...])


def _dense_layer2_body(cur, agg_a, agg_b, deg, s2, W1, b1, cur2_out):
    bn = deg.shape[0]
    rdeg = 1.0 / jnp.maximum(deg[...], 1.0)
    agg = jnp.concatenate([agg_a[...], agg_b[...]], axis=1)
    c = cur[...].reshape(bn, -1)
    h = jax.nn.relu((c + agg * rdeg) * 0.5)
    c2 = (jnp.dot(h, W1[...], preferred_element_type=jnp.float32) + b1[...]) * s2[...]
    cur2_out[...] = c2.reshape(cur2_out.shape)


def _combine_body(cur, agg_a, agg_b, deg, out):
    bn = deg.shape[0]
    rdeg = 1.0 / jnp.maximum(deg[...], 1.0)
    agg = jnp.concatenate([agg_a[...], agg_b[...]], axis=1)
    out[...] = (cur[...].reshape(bn, -1) + agg * rdeg) * 0.5


def _row_spec(bshape):
    nd = len(bshape)
    return pl.BlockSpec(bshape, lambda i, _nd=nd: (i,) + (0,) * (_nd - 1))


def _full_spec(shape):
    nd = len(shape)
    return pl.BlockSpec(shape, lambda i, _nd=nd: (0,) * _nd)


def _make_sc_aggregate(n, e, h):
    hh = h // NC
    ept = e // NS
    ch = 80
    nch = ept // ch
    assert ept % ch == 0 and n % NS == 0 and hh == 128
    rpt = n // NS
    zr = 125
    assert rpt % zr == 0
    degp = ((n + NS * L - 1) // (NS * L)) * NS * L
    rr = degp // NS

    mesh = plsc.VectorSubcoreMesh(core_axis_name="c", subcore_axis_name="s",
                                  num_cores=NC, num_subcores=NS)

    @functools.partial(
        pl.kernel,
        out_type=(
            jax.ShapeDtypeStruct((n, hh), jnp.float32),
            jax.ShapeDtypeStruct((n, hh), jnp.float32),
            jax.ShapeDtypeStruct((degp,), jnp.float32),
        ),
        mesh=mesh,
        scratch_types=[
            pltpu.VMEM((zr, hh), jnp.float32),
            pltpu.VMEM((ch,), jnp.int32),
            pltpu.VMEM((ch,), jnp.int32),
            pltpu.VMEM((ch,), jnp.int32),
            pltpu.VMEM((ch, hh), jnp.float32),
            pltpu.VMEM((degp,), jnp.float32),
            pltpu.VMEM((NS, rr), jnp.float32),
            pltpu.VMEM((rr,), jnp.float32),
            pltpu.VMEM_SHARED((n, hh), jnp.float32),
            pltpu.HBM((NS, degp), jnp.float32),
            pltpu.SemaphoreType.DMA,
        ],
        compiler_params=pltpu.CompilerParams(use_tc_tiling_on_sc=False,
                                             needs_layout_passes=False),
    )
    def sc_aggregate(row_hbm, col_hbm, cur_hbm, agg_a_hbm, agg_b_hbm, deg_hbm,
                     zbuf, idx_row, idx_col, idx2, rows, deg_v, red_buf,
                     deg_out_v, agg_sh, deg_sh, gsem):
        cid = lax.axis_index("c")
        sid = lax.axis_index("s")
        zeros16 = jnp.zeros((L,), jnp.float32)
        ones16 = jnp.ones((L,), jnp.float32)

        def _zrow(i, _):
            for j in range(hh // L):
                zbuf[i, pl.ds(j * L, L)] = zeros16
            return 0
        lax.fori_loop(0, zr, _zrow, 0)

        def _zdeg(i, _):
            deg_v[pl.ds(i * L, L)] = zeros16
            return 0
        lax.fori_loop(0, degp // L, _zdeg, 0)

        for j in range(rpt // zr):
            pltpu.sync_copy(zbuf, agg_sh.at[pl.ds(sid * rpt + j * zr, zr)])
        plsc.subcore_barrier()

        def _chunk(k, _):
            base = sid * ept + k * ch
            pltpu.sync_copy(row_hbm.at[pl.ds(base, ch)], idx_row)
            pltpu.sync_copy(col_hbm.at[pl.ds(base, ch)], idx_col)
            for j in range(ch // L):
                r = idx_row[pl.ds(j * L, L)]
                idx2[pl.ds(j * L, L)] = r + r + cid

            @pl.when(cid == 0)
            def _():
                for j in range(ch // L):
                    c = idx_col[pl.ds(j * L, L)]
                    plsc.addupdate_scatter(deg_v, [c], ones16)

            pltpu.async_copy(cur_hbm.at[idx2], rows, gsem).wait()
            pltpu.sync_copy(rows, agg_sh.at[idx_col], add=True)
            return 0
        lax.fori_loop(0, nch, _chunk, 0)
        plsc.subcore_barrier()

        @pl.when(cid == 0)
        def _():
            pltpu.sync_copy(agg_sh.at[pl.ds(sid * rpt, rpt)],
                            agg_a_hbm.at[pl.ds(sid * rpt, rpt)])
            pltpu.sync_copy(deg_v, deg_sh.at[sid])

        @pl.when(cid == 1)
        def _():
            pltpu.sync_copy(agg_sh.at[pl.ds(sid * rpt, rpt)],
                            agg_b_hbm.at[pl.ds(sid * rpt, rpt)])

        plsc.subcore_barrier()

        @pl.when(cid == 0)
        def _():
            for tt in range(NS):
                pltpu.sync_copy(deg_sh.at[tt, pl.ds(sid * rr, rr)], red_buf.at[tt])

            def _red(v, _):
                acc = red_buf[0, pl.ds(v * L, L)]
                for tt in range(1, NS):
                    acc = acc + red_buf[tt, pl.ds(v * L, L)]
                deg_out_v[pl.ds(v * L, L)] = acc
                return 0
            lax.fori_loop(0, rr // L, _red, 0)
            pltpu.sync_copy(deg_out_v, deg_hbm.at[pl.ds(sid * rr, rr)])

    return sc_aggregate, degp


def kernel(edge_index, node_features, node_texts, Wt, bt, Wg01, bg01, Wg02,
           bg02, Wg11, bg11, Wg12, bg12, W0, b0, W1, b1):
    n, d = node_features.shape
    t = node_texts.shape[1]
    h = W0.shape[1]
    e = edge_index.shape[1]
    assert n % BN == 0
    nblk = n // BN

    row = edge_index[0]
    col = edge_index[1]
    b2 = lambda b: b.reshape(1, -1)

    cur1, s2 = pl.pallas_call(
        _dense_layer1_body,
        grid=(nblk,),
        in_specs=[
            _row_spec((BN, t)), _row_spec((BN, d)),
            _full_spec((t, h)), _full_spec((1, h)),
            _full_spec((h, h)), _full_spec((1, h)),
            _full_spec((h, h)), _full_spec((1, h)),
            _full_spec((h, h)), _full_spec((1, h)),
            _full_spec((h, h)), _full_spec((1, h)),
            _full_spec((d, h)), _full_spec((1, h)),
        ],
        out_specs=[_row_spec((BN, 2, h // 2)), _row_spec((BN, h))],
        out_shape=[
            jax.ShapeDtypeStruct((n, 2, h // 2), jnp.float32),
            jax.ShapeDtypeStruct((n, h), jnp.float32),
        ],
    )(node_texts, node_features, Wt, b2(bt), Wg01, b2(bg01), Wg02, b2(bg02),
      Wg11, b2(bg11), Wg12, b2(bg12), W0, b2(b0))

    sc_aggregate, degp = _make_sc_aggregate(n, e, h)

    agg1_a, agg1_b, deg = sc_aggregate(row, col, cur1.reshape(2 * n, h // 2))
    deg2d = deg[:n].reshape(n, 1)

    cur2 = pl.pallas_call(
        _dense_layer2_body,
        grid=(nblk,),
        in_specs=[
            _row_spec((BN, 2, h // 2)), _row_spec((BN, h // 2)),
            _row_spec((BN, h // 2)), _row_spec((BN, 1)), _row_spec((BN, h)),
            _full_spec((h, h)), _full_spec((1, h)),
        ],
        out_specs=_row_spec((BN, 2, h // 2)),
        out_shape=jax.ShapeDtypeStruct((n, 2, h // 2), jnp.float32),
    )(cur1, agg1_a, agg1_b, deg2d, s2, W1, b2(b1))

    agg2_a, agg2_b, _ = sc_aggregate(row, col, cur2.reshape(2 * n, h // 2))

    out = pl.pallas_call(
        _combine_body,
        grid=(nblk,),
        in_specs=[
            _row_spec((BN, 2, h // 2)), _row_spec((BN, h // 2)),
            _row_spec((BN, h // 2)), _row_spec((BN, 1)),
        ],
        out_specs=_row_spec((BN, h)),
        out_shape=jax.ShapeDtypeStruct((n, h), jnp.float32),
    )(cur2, agg2_a, agg2_b, deg2d)
    return out


# speedup vs baseline: 5.9399x; 1.7180x over previous
import functools

import jax
import jax.numpy as jnp
from jax import lax
from jax.experimental import pallas as pl
from jax.experimental.pallas import tpu as pltpu
from jax.experimental.pallas import tpu_sc as plsc

NC = 2
NS = 16
L = 16

BN = 400


def _dense_layer1_body(texts, nf, Wt, bt, Wg01, bg01, Wg02, bg02,
                       Wg11, bg11, Wg12, bg12, W0, b0,
                       cur_out, s2_out):
    te = jnp.dot(texts[...], Wt[...], preferred_element_type=jnp.float32) + bt[...]
    t1 = jax.nn.relu(jnp.dot(te, Wg01[...], preferred_element_type=jnp.float32) + bg01[...])
    s1 = jax.nn.sigmoid(jnp.dot(t1, Wg02[...], preferred_element_type=jnp.float32) + bg02[...])
    c = (jnp.dot(nf[...], W0[...], preferred_element_type=jnp.float32) + b0[...]) * s1
    cur_out[...] = c.reshape(cur_out.shape)
    t2 = jax.nn.relu(jnp.dot(te, Wg11[...], preferred_element_type=jnp.float32) + bg11[...])
    s2_out[...] = jax.nn.sigmoid(jnp.dot(t2, Wg12[...], preferred_element_type=jnp.float32) + bg12[...])


def _dense_layer2_body(cur, agg_a, agg_b, deg, s2, W1, b1, cur2_out):
    bn = deg.shape[0]
    rdeg = 1.0 / jnp.maximum(deg[...], 1.0)
    agg = jnp.concatenate([agg_a[...], agg_b[...]], axis=1)
    c = cur[...].reshape(bn, -1)
    h = jax.nn.relu((c + agg * rdeg) * 0.5)
    c2 = (jnp.dot(h, W1[...], preferred_element_type=jnp.float32) + b1[...]) * s2[...]
    cur2_out[...] = c2.reshape(cur2_out.shape)


def _combine_body(cur, agg_a, agg_b, deg, out):
    bn = deg.shape[0]
    rdeg = 1.0 / jnp.maximum(deg[...], 1.0)
    agg = jnp.concatenate([agg_a[...], agg_b[...]], axis=1)
    out[...] = (cur[...].reshape(bn, -1) + agg * rdeg) * 0.5


def _row_spec(bshape):
    nd = len(bshape)
    return pl.BlockSpec(bshape, lambda i, _nd=nd: (i,) + (0,) * (_nd - 1))


def _full_spec(shape):
    nd = len(shape)
    return pl.BlockSpec(shape, lambda i, _nd=nd: (0,) * _nd)


def _make_sc_aggregate(n, e, h, compute_deg):
    hh = h // NC
    ch = 80
    sup = 10
    cpt = e // (NS * ch)
    nsup = cpt // sup
    assert e % (NS * ch) == 0 and cpt % sup == 0 and n % NS == 0 and hh == 128
    rpt = n // NS
    degp = ((n + NS * L - 1) // (NS * L)) * NS * L
    rr = degp // NS

    mesh = plsc.VectorSubcoreMesh(core_axis_name="c", subcore_axis_name="s",
                                  num_cores=NC, num_subcores=NS)

    out_type = [
        jax.ShapeDtypeStruct((n, hh), jnp.float32),
        jax.ShapeDtypeStruct((n, hh), jnp.float32),
    ]
    scratch = [
        pltpu.VMEM((sup, ch), jnp.int32),
        pltpu.VMEM((sup, ch), jnp.int32),
        pltpu.VMEM((ch, hh), jnp.float32),
        pltpu.VMEM((ch, hh), jnp.float32),
        pltpu.VMEM_SHARED((n, hh), jnp.float32),
        pltpu.SemaphoreType.DMA,
        pltpu.SemaphoreType.DMA,
        pltpu.SemaphoreType.DMA,
        pltpu.SemaphoreType.DMA,
    ]
    if compute_deg:
        out_type.append(jax.ShapeDtypeStruct((degp,), jnp.float32))
        scratch += [
            pltpu.VMEM((degp,), jnp.float32),
            pltpu.VMEM((NS, rr), jnp.float32),
            pltpu.VMEM((rr,), jnp.float32),
            pltpu.HBM((NS, degp), jnp.float32),
        ]

    @functools.partial(
        pl.kernel,
        out_type=tuple(out_type),
        mesh=mesh,
        scratch_types=scratch,
        compiler_params=pltpu.CompilerParams(use_tc_tiling_on_sc=False,
                                             needs_layout_passes=False),
    )
    def sc_aggregate(row_hbm, col_hbm, cur_hbm, agg_a_hbm, agg_b_hbm, *rest):
        if compute_deg:
            (deg_hbm, rowbuf, colbuf, rb0, rb1, agg_sh,
             g0, g1, s0, s1, deg_v, red_buf, deg_out_v, deg_sh) = rest
        else:
            (rowbuf, colbuf, rb0, rb1, agg_sh, g0, g1, s0, s1) = rest
        rb = (rb0, rb1)
        gsem = (g0, g1)
        ssem = (s0, s1)
        cid = lax.axis_index("c")
        sid = lax.axis_index("s")
        zeros16 = jnp.zeros((L,), jnp.float32)
        ones16 = jnp.ones((L,), jnp.float32)

        def _zrow(i, _):
            for j in range(hh // L):
                rb0[i, pl.ds(j * L, L)] = zeros16
            return 0
        lax.fori_loop(0, ch, _zrow, 0)
        done = 0
        while done < rpt:
            step = min(ch, rpt - done)
            pltpu.sync_copy(rb0.at[pl.ds(0, step)],
                            agg_sh.at[pl.ds(sid * rpt + done, step)])
            done += step
        if compute_deg:
            def _zdeg(i, _):
                deg_v[pl.ds(i * L, L)] = zeros16
                return 0
            lax.fori_loop(0, degp // L, _zdeg, 0)
        plsc.subcore_barrier()

        def _super(s, _):
            base = sid * cpt + s * sup
            pltpu.sync_copy(row_hbm.at[pl.ds(base, sup)], rowbuf)
            pltpu.sync_copy(col_hbm.at[pl.ds(base, sup)], colbuf)
            for j in range(sup):
                for l in range(ch // L):
                    r = rowbuf[j, pl.ds(l * L, L)]
                    rowbuf[j, pl.ds(l * L, L)] = r + r + cid
            if compute_deg:
                @pl.when(cid == 0)
                def _():
                    for j in range(sup):
                        for l in range(ch // L):
                            c = colbuf[j, pl.ds(l * L, L)]
                            plsc.addupdate_scatter(deg_v, [c], ones16)

            gd = [None, None]
            sd = [None, None]
            gd[0] = pltpu.async_copy(cur_hbm.at[rowbuf.at[0]], rb0, g0)
            for j in range(sup):
                b = j % 2
                gd[b].wait()
                if j + 1 < sup:
                    nb = (j + 1) % 2
                    if sd[nb] is not None:
                        sd[nb].wait()
                    gd[nb] = pltpu.async_copy(cur_hbm.at[rowbuf.at[j + 1]],
                                              rb[nb], gsem[nb])
                sd[b] = pltpu.async_copy(rb[b], agg_sh.at[colbuf.at[j]],
                                         ssem[b], add=True)
            sd[0].wait()
            sd[1].wait()
            return 0
        lax.fori_loop(0, nsup, _super, 0)
        plsc.subcore_barrier()

        @pl.when(cid == 0)
        def _():
            pltpu.sync_copy(agg_sh.at[pl.ds(sid * rpt, rpt)],
                            agg_a_hbm.at[pl.ds(sid * rpt, rpt)])

        @pl.when(cid == 1)
        def _():
            pltpu.sync_copy(agg_sh.at[pl.ds(sid * rpt, rpt)],
                            agg_b_hbm.at[pl.ds(sid * rpt, rpt)])

        if compute_deg:
            @pl.when(cid == 0)
            def _():
                pltpu.sync_copy(deg_v, deg_sh.at[sid])
            plsc.subcore_barrier()

            @pl.when(cid == 0)
            def _():
                for tt in range(NS):
                    pltpu.sync_copy(deg_sh.at[tt, pl.ds(sid * rr, rr)],
                                    red_buf.at[tt])

                def _red(v, _):
                    acc = red_buf[0, pl.ds(v * L, L)]
                    for tt in range(1, NS):
                        acc = acc + red_buf[tt, pl.ds(v * L, L)]
                    deg_out_v[pl.ds(v * L, L)] = acc
                    return 0
                lax.fori_loop(0, rr // L, _red, 0)
                pltpu.sync_copy(deg_out_v, deg_hbm.at[pl.ds(sid * rr, rr)])

    return sc_aggregate, degp, ch


def kernel(edge_index, node_features, node_texts, Wt, bt, Wg01, bg01, Wg02,
           bg02, Wg11, bg11, Wg12, bg12, W0, b0, W1, b1):
    n, d = node_features.shape
    t = node_texts.shape[1]
    h = W0.shape[1]
    e = edge_index.shape[1]
    assert n % BN == 0
    nblk = n // BN

    row = edge_index[0]
    col = edge_index[1]
    b2 = lambda b: b.reshape(1, -1)

    cur1, s2 = pl.pallas_call(
        _dense_layer1_body,
        grid=(nblk,),
        in_specs=[
            _row_spec((BN, t)), _row_spec((BN, d)),
            _full_spec((t, h)), _full_spec((1, h)),
            _full_spec((h, h)), _full_spec((1, h)),
            _full_spec((h, h)), _full_spec((1, h)),
            _full_spec((h, h)), _full_spec((1, h)),
            _full_spec((h, h)), _full_spec((1, h)),
            _full_spec((d, h)), _full_spec((1, h)),
        ],
        out_specs=[_row_spec((BN, 2, h // 2)), _row_spec((BN, h))],
        out_shape=[
            jax.ShapeDtypeStruct((n, 2, h // 2), jnp.float32),
            jax.ShapeDtypeStruct((n, h), jnp.float32),
        ],
    )(node_texts, node_features, Wt, b2(bt), Wg01, b2(bg01), Wg02, b2(bg02),
      Wg11, b2(bg11), Wg12, b2(bg12), W0, b2(b0))

    sc_aggregate1, degp, ch = _make_sc_aggregate(n, e, h, compute_deg=True)
    sc_aggregate2, _, _ = _make_sc_aggregate(n, e, h, compute_deg=False)
    row2d = row.reshape(e // ch, ch)
    col2d = col.reshape(e // ch, ch)

    agg1_a, agg1_b, deg = sc_aggregate1(row2d, col2d,
                                        cur1.reshape(2 * n, h // 2))
    deg2d = deg[:n].reshape(n, 1)

    cur2 = pl.pallas_call(
        _dense_layer2_body,
        grid=(nblk,),
        in_specs=[
            _row_spec((BN, 2, h // 2)), _row_spec((BN, h // 2)),
            _row_spec((BN, h // 2)), _row_spec((BN, 1)), _row_spec((BN, h)),
            _full_spec((h, h)), _full_spec((1, h)),
        ],
        out_specs=_row_spec((BN, 2, h // 2)),
        out_shape=jax.ShapeDtypeStruct((n, 2, h // 2), jnp.float32),
    )(cur1, agg1_a, agg1_b, deg2d, s2, W1, b2(b1))

    agg2_a, agg2_b = sc_aggregate2(row2d, col2d, cur2.reshape(2 * n, h // 2))

    out = pl.pallas_call(
        _combine_body,
        grid=(nblk,),
        in_specs=[
            _row_spec((BN, 2, h // 2)), _row_spec((BN, h // 2)),
            _row_spec((BN, h // 2)), _row_spec((BN, 1)),
        ],
        out_specs=_row_spec((BN, h)),
        out_shape=jax.ShapeDtypeStruct((n, h), jnp.float32),
    )(cur2, agg2_a, agg2_b, deg2d)
    return out


# speedup vs baseline: 7.9597x; 1.3400x over previous
import functools

import jax
import jax.numpy as jnp
from jax import lax
from jax.experimental import pallas as pl
from jax.experimental.pallas import tpu as pltpu
from jax.experimental.pallas import tpu_sc as plsc

NC = 2
NS = 16
L = 16

BN = 400


def _dense_layer1_body(texts, nf, Wt, bt, Wg01, bg01, Wg02, bg02,
                       Wg11, bg11, Wg12, bg12, W0, b0,
                       cur_out, s2_out):
    te = jnp.dot(texts[...], Wt[...], preferred_element_type=jnp.float32) + bt[...]
    t1 = jax.nn.relu(jnp.dot(te, Wg01[...], preferred_element_type=jnp.float32) + bg01[...])
    s1 = jax.nn.sigmoid(jnp.dot(t1, Wg02[...], preferred_element_type=jnp.float32) + bg02[...])
    c = (jnp.dot(nf[...], W0[...], preferred_element_type=jnp.float32) + b0[...]) * s1
    cur_out[...] = c.reshape(cur_out.shape)
    t2 = jax.nn.relu(jnp.dot(te, Wg11[...], preferred_element_type=jnp.float32) + bg11[...])
    s2_out[...] = jax.nn.sigmoid(jnp.dot(t2, Wg12[...], preferred_element_type=jnp.float32) + bg12[...])


def _dense_layer2_body(cur, agg_a, agg_b, deg, s2, W1, b1, cur2_out):
    bn = deg.shape[0]
    rdeg = 1.0 / jnp.maximum(deg[...], 1.0)
    agg = jnp.concatenate([agg_a[...], agg_b[...]], axis=1)
    c = cur[...].reshape(bn, -1)
    h = jax.nn.relu((c + agg * rdeg) * 0.5)
    c2 = (jnp.dot(h, W1[...], preferred_element_type=jnp.float32) + b1[...]) * s2[...]
    cur2_out[...] = c2.reshape(cur2_out.shape)


def _combine_body(cur, agg_a, agg_b, deg, out):
    bn = deg.shape[0]
    rdeg = 1.0 / jnp.maximum(deg[...], 1.0)
    agg = jnp.concatenate([agg_a[...], agg_b[...]], axis=1)
    out[...] = (cur[...].reshape(bn, -1) + agg * rdeg) * 0.5


def _row_spec(bshape):
    nd = len(bshape)
    return pl.BlockSpec(bshape, lambda i, _nd=nd: (i,) + (0,) * (_nd - 1))


def _full_spec(shape):
    nd = len(shape)
    return pl.BlockSpec(shape, lambda i, _nd=nd: (0,) * _nd)


def _make_sc_aggregate(n, e, h, compute_deg):
    hh = h // NC
    ch = 80
    sup = 10
    cpt = e // (NS * ch)
    nsup = cpt // sup
    assert e % (NS * ch) == 0 and cpt % sup == 0 and n % NS == 0 and hh == 128
    rpt = n // NS
    degp = ((n + NS * hh - 1) // (NS * hh)) * NS * hh
    dgr = degp // hh
    drt = dgr // NS
    nbuf = 3

    mesh = plsc.VectorSubcoreMesh(core_axis_name="c", subcore_axis_name="s",
                                  num_cores=NC, num_subcores=NS)

    out_type = [
        jax.ShapeDtypeStruct((n, hh), jnp.float32),
        jax.ShapeDtypeStruct((n, hh), jnp.float32),
    ]
    scratch = (
        [pltpu.VMEM((sup, ch), jnp.int32),
         pltpu.VMEM((sup, ch), jnp.int32)]
        + [pltpu.VMEM((ch, hh), jnp.float32) for _ in range(nbuf)]
        + [pltpu.VMEM_SHARED((n, hh), jnp.float32)]
        + [pltpu.SemaphoreType.DMA for _ in range(2 * nbuf)]
    )
    if compute_deg:
        out_type.append(jax.ShapeDtypeStruct((dgr, hh), jnp.float32))
        scratch += [
            pltpu.VMEM((dgr, hh), jnp.float32),
            pltpu.HBM((NS * dgr, hh), jnp.float32),
        ]

    @functools.partial(
        pl.kernel,
        out_type=tuple(out_type),
        mesh=mesh,
        scratch_types=scratch,
        compiler_params=pltpu.CompilerParams(use_tc_tiling_on_sc=False,
                                             needs_layout_passes=False),
    )
    def sc_aggregate(row_hbm, col_hbm, cur_hbm, agg_a_hbm, agg_b_hbm, *rest):
        if compute_deg:
            deg_hbm = rest[0]
            rest = rest[1:]
        rowbuf, colbuf = rest[0], rest[1]
        rb = rest[2:2 + nbuf]
        agg_sh = rest[2 + nbuf]
        gsem = rest[3 + nbuf:3 + 2 * nbuf]
        ssem = rest[3 + 2 * nbuf:3 + 3 * nbuf]
        if compute_deg:
            deg_v, deg_sh = rest[3 + 3 * nbuf], rest[4 + 3 * nbuf]
        cid = lax.axis_index("c")
        sid = lax.axis_index("s")
        zeros16 = jnp.zeros((L,), jnp.float32)
        ones16 = jnp.ones((L,), jnp.float32)

        def _zrow(i, _):
            for j in range(hh // L):
                rb[0][i, pl.ds(j * L, L)] = zeros16
            return 0
        lax.fori_loop(0, ch, _zrow, 0)
        done = 0
        while done < rpt:
            step = min(ch, rpt - done)
            pltpu.sync_copy(rb[0].at[pl.ds(0, step)],
                            agg_sh.at[pl.ds(sid * rpt + done, step)])
            done += step
        if compute_deg:
            def _zdeg(i, _):
                for j in range(hh // L):
                    deg_v[i, pl.ds(j * L, L)] = zeros16
                return 0
            lax.fori_loop(0, dgr, _zdeg, 0)
        plsc.subcore_barrier()

        def _super(s, _):
            base = sid * cpt + s * sup
            pltpu.sync_copy(row_hbm.at[pl.ds(base, sup)], rowbuf)
            pltpu.sync_copy(col_hbm.at[pl.ds(base, sup)], colbuf)
            for j in range(sup):
                for l in range(ch // L):
                    r = rowbuf[j, pl.ds(l * L, L)]
                    rowbuf[j, pl.ds(l * L, L)] = r + r + cid
            if compute_deg:
                @pl.when(cid == 0)
                def _():
                    for j in range(sup):
                        for l in range(ch // L):
                            c = colbuf[j, pl.ds(l * L, L)]
                            plsc.addupdate_scatter(
                                deg_v,
                                [lax.shift_right_logical(c, 7),
                                 lax.bitwise_and(c, 127)],
                                ones16)

            gd = [None] * nbuf
            sd = [None] * nbuf
            for j in range(min(nbuf - 1, sup)):
                gd[j] = pltpu.async_copy(cur_hbm.at[rowbuf.at[j]], rb[j],
                                         gsem[j])
            for j in range(sup):
                b = j % nbuf
                gd[b].wait()
                if j + nbuf - 1 < sup:
                    nb = (j + nbuf - 1) % nbuf
                    if sd[nb] is not None:
                        sd[nb].wait()
                    gd[nb] = pltpu.async_copy(
                        cur_hbm.at[rowbuf.at[j + nbuf - 1]], rb[nb], gsem[nb])
                sd[b] = pltpu.async_copy(rb[b], agg_sh.at[colbuf.at[j]],
                                         ssem[b], add=True)
            for b in range(nbuf):
                if sd[b] is not None:
                    sd[b].wait()
            return 0
        lax.fori_loop(0, nsup, _super, 0)
        plsc.subcore_barrier()

        @pl.when(cid == 0)
        def _():
            pltpu.sync_copy(agg_sh.at[pl.ds(sid * rpt, rpt)],
                            agg_a_hbm.at[pl.ds(sid * rpt, rpt)])

        @pl.when(cid == 1)
        def _():
            pltpu.sync_copy(agg_sh.at[pl.ds(sid * rpt, rpt)],
                            agg_b_hbm.at[pl.ds(sid * rpt, rpt)])

        if compute_deg:
            @pl.when(cid == 0)
            def _():
                pltpu.sync_copy(deg_v, deg_sh.at[pl.ds(sid * dgr, dgr)])
            plsc.subcore_barrier()

            @pl.when(cid == 0)
            def _():
                for g in range(2):
                    for t8 in range(NS // 2):
                        tt = g * (NS // 2) + t8
                        pltpu.sync_copy(
                            deg_sh.at[pl.ds(tt * dgr + sid * drt, drt)],
                            rb[0].at[pl.ds(t8 * drt, drt)])
                    for p in range(drt):
                        for q in range(hh // L):
                            acc = rb[0][p, pl.ds(q * L, L)]
                            for t8 in range(1, NS // 2):
                                acc = acc + rb[0][t8 * drt + p, pl.ds(q * L, L)]
                            if g == 0:
                                rb[1][p, pl.ds(q * L, L)] = acc
                            else:
                                rb[1][p, pl.ds(q * L, L)] = (
                                    rb[1][p, pl.ds(q * L, L)] + acc)
                pltpu.sync_copy(rb[1].at[pl.ds(0, drt)],
                                deg_hbm.at[pl.ds(sid * drt, drt)])

    return sc_aggregate, degp, ch


def kernel(edge_index, node_features, node_texts, Wt, bt, Wg01, bg01, Wg02,
           bg02, Wg11, bg11, Wg12, bg12, W0, b0, W1, b1):
    n, d = node_features.shape
    t = node_texts.shape[1]
    h = W0.shape[1]
    e = edge_index.shape[1]
    assert n % BN == 0
    nblk = n // BN

    row = edge_index[0]
    col = edge_index[1]
    b2 = lambda b: b.reshape(1, -1)

    cur1, s2 = pl.pallas_call(
        _dense_layer1_body,
        grid=(nblk,),
        in_specs=[
            _row_spec((BN, t)), _row_spec((BN, d)),
            _full_spec((t, h)), _full_spec((1, h)),
            _full_spec((h, h)), _full_spec((1, h)),
            _full_spec((h, h)), _full_spec((1, h)),
            _full_spec((h, h)), _full_spec((1, h)),
            _full_spec((h, h)), _full_spec((1, h)),
            _full_spec((d, h)), _full_spec((1, h)),
        ],
        out_specs=[_row_spec((BN, 2, h // 2)), _row_spec((BN, h))],
        out_shape=[
            jax.ShapeDtypeStruct((n, 2, h // 2), jnp.float32),
            jax.ShapeDtypeStruct((n, h), jnp.float32),
        ],
    )(node_texts, node_features, Wt, b2(bt), Wg01, b2(bg01), Wg02, b2(bg02),
      Wg11, b2(bg11), Wg12, b2(bg12), W0, b2(b0))

    sc_aggregate1, degp, ch = _make_sc_aggregate(n, e, h, compute_deg=True)
    sc_aggregate2, _, _ = _make_sc_aggregate(n, e, h, compute_deg=False)
    row2d = row.reshape(e // ch, ch)
    col2d = col.reshape(e // ch, ch)

    agg1_a, agg1_b, deg = sc_aggregate1(row2d, col2d,
                                        cur1.reshape(2 * n, h // 2))
    deg2d = deg.reshape(-1)[:n].reshape(n, 1)

    cur2 = pl.pallas_call(
        _dense_layer2_body,
        grid=(nblk,),
        in_specs=[
            _row_spec((BN, 2, h // 2)), _row_spec((BN, h // 2)),
            _row_spec((BN, h // 2)), _row_spec((BN, 1)), _row_spec((BN, h)),
            _full_spec((h, h)), _full_spec((1, h)),
        ],
        out_specs=_row_spec((BN, 2, h // 2)),
        out_shape=jax.ShapeDtypeStruct((n, 2, h // 2), jnp.float32),
    )(cur1, agg1_a, agg1_b, deg2d, s2, W1, b2(b1))

    agg2_a, agg2_b = sc_aggregate2(row2d, col2d, cur2.reshape(2 * n, h // 2))

    out = pl.pallas_call(
        _combine_body,
        grid=(nblk,),
        in_specs=[
            _row_spec((BN, 2, h // 2)), _row_spec((BN, h // 2)),
            _row_spec((BN, h // 2)), _row_spec((BN, 1)),
        ],
        out_specs=_row_spec((BN, h)),
        out_shape=jax.ShapeDtypeStruct((n, h), jnp.float32),
    )(cur2, agg2_a, agg2_b, deg2d)
    return out
